# baseline (speedup 1.0000x reference)
import jax
import jax.numpy as jnp
import numpy as np
from jax import lax
from jax.experimental import pallas as pl
from jax.experimental.pallas import tpu as pltpu
from jax.experimental.pallas import tpu_sc as plsc

N_VOCAB = 100000
EMBED_DIM = 768
BATCH = 4
SEQ_LEN = 8192

NUM_WORKERS = 32
POS_PER_WORKER = SEQ_LEN // NUM_WORKERS
CHUNK = 32
N_CHUNKS = POS_PER_WORKER // CHUNK
N_STEPS = N_CHUNKS * BATCH
N_BODY = 8
N_ITERS = N_STEPS // N_BODY
NRING = 4
LANES = 16
VECS_PER_ROW = EMBED_DIM // LANES
ADD_GROUP = 8


def _positional_encoding_np(max_len, d):
    pos = np.arange(max_len, dtype=np.float64)[:, None]
    i = np.arange(0, d, 2, dtype=np.float64)
    div = np.exp(-(np.log(10000.0) * i / d))
    ang = pos * div[None, :]
    pe = np.zeros((max_len, d), dtype=np.float64)
    pe[:, 0::2] = np.sin(ang)
    pe[:, 1::2] = np.cos(ang)
    return pe.astype(np.float32)


def _pack_pe_bf16(pe):
    u = pe.view(np.uint32).reshape(pe.shape[0], EMBED_DIM // 32, 2, LANES)
    bf = ((u + 0x7FFF + ((u >> 16) & 1)) >> 16).astype(np.uint32)
    packed = bf[:, :, 0, :] | (bf[:, :, 1, :] << 16)
    return packed.reshape(pe.shape[0], EMBED_DIM // 2).view(np.int32)


_PE_PACKED = _pack_pe_bf16(_positional_encoding_np(SEQ_LEN, EMBED_DIM))
PE_WORDS = EMBED_DIM // 2
PAIRS_PER_ROW = EMBED_DIM // 32
_HI_MASK = np.int32(-65536)


def _sc_body(x_hbm, table_hbm, pe_hbm, out_hbm, idx_all,
             pe0, pe1, r0, r1, r2, r3,
             g0, g1, g2, g3, s0, s1, s2, s3, p0, p1):
    rows = [r0, r1, r2, r3]
    gsem = [g0, g1, g2, g3]
    ssem = [s0, s1, s2, s3]
    pes = [pe0, pe1]
    pesem = [p0, p1]

    wid = lax.axis_index("s") * 2 + lax.axis_index("c")
    pos0 = wid * POS_PER_WORKER

    for b in range(BATCH):
        pltpu.sync_copy(x_hbm.at[pl.ds(b * SEQ_LEN + pos0, POS_PER_WORKER)],
                        idx_all.at[b])

    def gcopy(t, slot):
        j, b = t // BATCH, t % BATCH
        return pltpu.make_async_copy(
            table_hbm.at[idx_all.at[b, pl.ds(j * CHUNK, CHUNK)]],
            rows[slot], gsem[slot])

    def scopy(t, slot):
        j, b = t // BATCH, t % BATCH
        base = b * SEQ_LEN + pos0 + j * CHUNK
        return pltpu.make_async_copy(
            rows[slot], out_hbm.at[pl.ds(base, CHUNK)], ssem[slot])

    def pecopy(j, par):
        return pltpu.make_async_copy(
            pe_hbm.at[pl.ds(pos0 + j * CHUNK, CHUNK)], pes[par], pesem[par])

    def add_chunk(rbuf, pbuf):
        shift16 = jnp.full((LANES,), 16, dtype=jnp.int32)
        himask = jnp.full((LANES,), _HI_MASK, dtype=jnp.int32)

        def add_row(r, c):
            for g in range(0, PAIRS_PER_ROW, ADD_GROUP // 2):
                acc = []
                for m in range(g, g + ADD_GROUP // 2):
                    w = pbuf[r, pl.ds(m * LANES, LANES)]
                    lo = lax.bitcast_convert_type(lax.shift_left(w, shift16), jnp.float32)
                    hi = lax.bitcast_convert_type(jnp.bitwise_and(w, himask), jnp.float32)
                    sl_lo = pl.ds(2 * m * LANES, LANES)
                    sl_hi = pl.ds((2 * m + 1) * LANES, LANES)
                    acc.append((sl_lo, rbuf[r, sl_lo] + lo))
                    acc.append((sl_hi, rbuf[r, sl_hi] + hi))
                for sl, v in acc:
                    rbuf[r, sl] = v
            return c

        lax.fori_loop(0, CHUNK, add_row, 0)

    pecopy(0, 0).start()
    pecopy(1, 1).start()
    gcopy(0, 0).start()
    gcopy(1, 1).start()

    def body(i, carry):
        for u in range(N_BODY):
            t = N_BODY * i + u
            nslot = (u + 2) % NRING
            if u < 2:
                @pl.when(i >= 1)
                def _ws():
                    scopy(t - 2, nslot).wait()
            else:
                scopy(t - 2, nslot).wait()
            if u >= N_BODY - 2:
                @pl.when(i + 1 < N_ITERS)
                def _g():
                    gcopy(t + 2, nslot).start()
            else:
                gcopy(t + 2, nslot).start()

            if u == 0:
                @pl.when(i >= 1)
                def _p1():
                    pecopy(2 * i + 1, 1).start()
                pecopy(2 * i, 0).wait()
            elif u == 4:
                @pl.when(i + 1 < N_ITERS)
                def _p0():
                    pecopy(2 * i + 2, 0).start()
                pecopy(2 * i + 1, 1).wait()

            gcopy(t, u % NRING).wait()
            add_chunk(rows[u % NRING], pes[0] if u < 4 else pes[1])
            scopy(t, u % NRING).start()
        return carry

    lax.fori_loop(0, N_ITERS, body, 0)
    scopy(N_STEPS - 2, (N_STEPS - 2) % NRING).wait()
    scopy(N_STEPS - 1, (N_STEPS - 1) % NRING).wait()


def kernel(x, token_table):
    x_flat = x.reshape(-1).astype(jnp.int32)
    pe = jnp.asarray(_PE_PACKED)

    mesh = plsc.VectorSubcoreMesh(core_axis_name="c", subcore_axis_name="s")
    run = pl.kernel(
        _sc_body,
        out_type=jax.ShapeDtypeStruct((BATCH * SEQ_LEN, EMBED_DIM), jnp.float32),
        mesh=mesh,
        scratch_types=(
            [pltpu.VMEM((BATCH, POS_PER_WORKER), jnp.int32)]
            + [pltpu.VMEM((CHUNK, PE_WORDS), jnp.int32)] * 2
            + [pltpu.VMEM((CHUNK, EMBED_DIM), jnp.float32)] * 4
            + [pltpu.SemaphoreType.DMA] * 10
        ),
    )
    out = run(x_flat, token_table, pe)
    return out.reshape(BATCH, SEQ_LEN, EMBED_DIM)


# speedup vs baseline: 1.0271x; 1.0271x over previous
import jax
import jax.numpy as jnp
import numpy as np
from jax import lax
from jax.experimental import pallas as pl
from jax.experimental.pallas import tpu as pltpu
from jax.experimental.pallas import tpu_sc as plsc

N_VOCAB = 100000
EMBED_DIM = 768
BATCH = 4
SEQ_LEN = 8192

NUM_WORKERS = 32
POS_PER_WORKER = SEQ_LEN // NUM_WORKERS
CHUNK = 16
N_CHUNKS = POS_PER_WORKER // CHUNK
N_STEPS = N_CHUNKS * BATCH
N_BODY = 8
N_ITERS = N_STEPS // N_BODY
NRING = 8
LOOKAHEAD = 4
LANES = 16
VECS_PER_ROW = EMBED_DIM // LANES
ADD_GROUP = 8


def _positional_encoding_np(max_len, d):
    pos = np.arange(max_len, dtype=np.float64)[:, None]
    i = np.arange(0, d, 2, dtype=np.float64)
    div = np.exp(-(np.log(10000.0) * i / d))
    ang = pos * div[None, :]
    pe = np.zeros((max_len, d), dtype=np.float64)
    pe[:, 0::2] = np.sin(ang)
    pe[:, 1::2] = np.cos(ang)
    return pe.astype(np.float32)


def _pack_pe_bf16(pe):
    u = pe.view(np.uint32).reshape(pe.shape[0], EMBED_DIM // 32, 2, LANES)
    bf = ((u + 0x7FFF + ((u >> 16) & 1)) >> 16).astype(np.uint32)
    packed = bf[:, :, 0, :] | (bf[:, :, 1, :] << 16)
    return packed.reshape(pe.shape[0], EMBED_DIM // 2).view(np.int32)


_PE_PACKED = _pack_pe_bf16(_positional_encoding_np(SEQ_LEN, EMBED_DIM))
PE_WORDS = EMBED_DIM // 2
PAIRS_PER_ROW = EMBED_DIM // 32
_HI_MASK = np.int32(-65536)


def _sc_body(x_hbm, table_hbm, pe_hbm, out_hbm, idx_all,
             pe0, pe1, r0, r1, r2, r3, r4, r5, r6, r7,
             g0, g1, g2, g3, g4, g5, g6, g7,
             s0, s1, s2, s3, s4, s5, s6, s7, p0, p1):
    rows = [r0, r1, r2, r3, r4, r5, r6, r7]
    gsem = [g0, g1, g2, g3, g4, g5, g6, g7]
    ssem = [s0, s1, s2, s3, s4, s5, s6, s7]
    pes = [pe0, pe1]
    pesem = [p0, p1]

    wid = lax.axis_index("s") * 2 + lax.axis_index("c")
    pos0 = wid * POS_PER_WORKER

    for b in range(BATCH):
        pltpu.sync_copy(x_hbm.at[pl.ds(b * SEQ_LEN + pos0, POS_PER_WORKER)],
                        idx_all.at[b])

    def gcopy(t, slot):
        j, b = t // BATCH, t % BATCH
        return pltpu.make_async_copy(
            table_hbm.at[idx_all.at[b, pl.ds(j * CHUNK, CHUNK)]],
            rows[slot], gsem[slot])

    def scopy(t, slot):
        j, b = t // BATCH, t % BATCH
        base = b * SEQ_LEN + pos0 + j * CHUNK
        return pltpu.make_async_copy(
            rows[slot], out_hbm.at[pl.ds(base, CHUNK)], ssem[slot])

    def pecopy(j, par):
        return pltpu.make_async_copy(
            pe_hbm.at[pl.ds(pos0 + j * CHUNK, CHUNK)], pes[par], pesem[par])

    def add_chunk(rbuf, pbuf):
        shift16 = jnp.full((LANES,), 16, dtype=jnp.int32)
        himask = jnp.full((LANES,), _HI_MASK, dtype=jnp.int32)

        def add_row(r, c):
            for g in range(0, PAIRS_PER_ROW, ADD_GROUP // 2):
                acc = []
                for m in range(g, g + ADD_GROUP // 2):
                    w = pbuf[r, pl.ds(m * LANES, LANES)]
                    lo = lax.bitcast_convert_type(lax.shift_left(w, shift16), jnp.float32)
                    hi = lax.bitcast_convert_type(jnp.bitwise_and(w, himask), jnp.float32)
                    sl_lo = pl.ds(2 * m * LANES, LANES)
                    sl_hi = pl.ds((2 * m + 1) * LANES, LANES)
                    acc.append((sl_lo, rbuf[r, sl_lo] + lo))
                    acc.append((sl_hi, rbuf[r, sl_hi] + hi))
                for sl, v in acc:
                    rbuf[r, sl] = v
            return c

        lax.fori_loop(0, CHUNK, add_row, 0)

    pecopy(0, 0).start()
    pecopy(1, 1).start()
    for t0 in range(LOOKAHEAD):
        gcopy(t0, t0 % NRING).start()

    def body(i, carry):
        for u in range(N_BODY):
            t = N_BODY * i + u
            nslot = (u + LOOKAHEAD) % NRING
            if u < NRING - LOOKAHEAD:
                @pl.when(i >= 1)
                def _ws():
                    scopy(t - (NRING - LOOKAHEAD), nslot).wait()
            else:
                scopy(t - (NRING - LOOKAHEAD), nslot).wait()
            if u >= N_BODY - LOOKAHEAD:
                @pl.when(i + 1 < N_ITERS)
                def _g():
                    gcopy(t + LOOKAHEAD, nslot).start()
            else:
                gcopy(t + LOOKAHEAD, nslot).start()

            if u == 0:
                @pl.when(i >= 1)
                def _p1():
                    pecopy(2 * i + 1, 1).start()
                pecopy(2 * i, 0).wait()
            elif u == 4:
                @pl.when(i + 1 < N_ITERS)
                def _p0():
                    pecopy(2 * i + 2, 0).start()
                pecopy(2 * i + 1, 1).wait()

            gcopy(t, u % NRING).wait()
            add_chunk(rows[u % NRING], pes[0] if u < 4 else pes[1])
            scopy(t, u % NRING).start()
        return carry

    lax.fori_loop(0, N_ITERS, body, 0)
    for tl in range(N_STEPS - (NRING - LOOKAHEAD), N_STEPS):
        scopy(tl, tl % NRING).wait()


def kernel(x, token_table):
    x_flat = x.reshape(-1).astype(jnp.int32)
    pe = jnp.asarray(_PE_PACKED)

    mesh = plsc.VectorSubcoreMesh(core_axis_name="c", subcore_axis_name="s")
    run = pl.kernel(
        _sc_body,
        out_type=jax.ShapeDtypeStruct((BATCH * SEQ_LEN, EMBED_DIM), jnp.float32),
        mesh=mesh,
        scratch_types=(
            [pltpu.VMEM((BATCH, POS_PER_WORKER), jnp.int32)]
            + [pltpu.VMEM((CHUNK, PE_WORDS), jnp.int32)] * 2
            + [pltpu.VMEM((CHUNK, EMBED_DIM), jnp.float32)] * NRING
            + [pltpu.SemaphoreType.DMA] * (2 * NRING + 2)
        ),
    )
    out = run(x_flat, token_table, pe)
    return out.reshape(BATCH, SEQ_LEN, EMBED_DIM)


# speedup vs baseline: 1.0429x; 1.0154x over previous
import jax
import jax.numpy as jnp
import numpy as np
from jax import lax
from jax.experimental import pallas as pl
from jax.experimental.pallas import tpu as pltpu
from jax.experimental.pallas import tpu_sc as plsc

N_VOCAB = 100000
EMBED_DIM = 768
BATCH = 4
SEQ_LEN = 8192

NUM_WORKERS = 32
POS_PER_WORKER = SEQ_LEN // NUM_WORKERS
CHUNK = 16
N_CHUNKS = POS_PER_WORKER // CHUNK
N_STEPS = N_CHUNKS * BATCH
N_BODY = 8
N_ITERS = N_STEPS // N_BODY
NRING = 8
LOOKAHEAD = 4
LANES = 16
VECS_PER_ROW = EMBED_DIM // LANES
ADD_GROUP = 8


def _positional_encoding_np(max_len, d):
    pos = np.arange(max_len, dtype=np.float64)[:, None]
    i = np.arange(0, d, 2, dtype=np.float64)
    div = np.exp(-(np.log(10000.0) * i / d))
    ang = pos * div[None, :]
    pe = np.zeros((max_len, d), dtype=np.float64)
    pe[:, 0::2] = np.sin(ang)
    pe[:, 1::2] = np.cos(ang)
    return pe.astype(np.float32)


def _pack_pe_bf16(pe):
    u = pe.view(np.uint32).reshape(pe.shape[0], EMBED_DIM // 32, 2, LANES)
    bf = ((u + 0x7FFF + ((u >> 16) & 1)) >> 16).astype(np.uint32)
    packed = bf[:, :, 0, :] | (bf[:, :, 1, :] << 16)
    return packed.reshape(pe.shape[0], EMBED_DIM // 2).view(np.int32)


_PE_PACKED = _pack_pe_bf16(_positional_encoding_np(SEQ_LEN, EMBED_DIM))
PE_WORDS = EMBED_DIM // 2
PAIRS_PER_ROW = EMBED_DIM // 32
_HI_MASK = np.int32(-65536)


def _sc_body(x_hbm, table_hbm, pe_hbm, out_hbm, idx_all,
             pe0, pe1, r0, r1, r2, r3, r4, r5, r6, r7,
             g0, g1, g2, g3, g4, g5, g6, g7,
             s0, s1, s2, s3, s4, s5, s6, s7, p0, p1):
    rows = [r0, r1, r2, r3, r4, r5, r6, r7]
    gsem = [g0, g1, g2, g3, g4, g5, g6, g7]
    ssem = [s0, s1, s2, s3, s4, s5, s6, s7]
    pes = [pe0, pe1]
    pesem = [p0, p1]

    wid = lax.axis_index("s") * 2 + lax.axis_index("c")
    pos0 = wid * POS_PER_WORKER

    for b in range(BATCH):
        pltpu.sync_copy(x_hbm.at[b, pl.ds(pos0, POS_PER_WORKER)],
                        idx_all.at[b])

    def gcopy(t, slot):
        j, b = t // BATCH, t % BATCH
        return pltpu.make_async_copy(
            table_hbm.at[idx_all.at[b, pl.ds(j * CHUNK, CHUNK)]],
            rows[slot], gsem[slot])

    def scopy(t, slot):
        j, b = t // BATCH, t % BATCH
        base = b * SEQ_LEN + pos0 + j * CHUNK
        return pltpu.make_async_copy(
            rows[slot], out_hbm.at[pl.ds(base, CHUNK)], ssem[slot])

    def pecopy(j, par):
        return pltpu.make_async_copy(
            pe_hbm.at[pl.ds(pos0 + j * CHUNK, CHUNK)], pes[par], pesem[par])

    def add_chunk(rbuf, pbuf):
        shift16 = jnp.full((LANES,), 16, dtype=jnp.int32)
        himask = jnp.full((LANES,), _HI_MASK, dtype=jnp.int32)

        def add_row(r, c):
            for g in range(0, PAIRS_PER_ROW, ADD_GROUP // 2):
                acc = []
                for m in range(g, g + ADD_GROUP // 2):
                    w = pbuf[r, pl.ds(m * LANES, LANES)]
                    lo = lax.bitcast_convert_type(lax.shift_left(w, shift16), jnp.float32)
                    hi = lax.bitcast_convert_type(jnp.bitwise_and(w, himask), jnp.float32)
                    sl_lo = pl.ds(2 * m * LANES, LANES)
                    sl_hi = pl.ds((2 * m + 1) * LANES, LANES)
                    acc.append((sl_lo, rbuf[r, sl_lo] + lo))
                    acc.append((sl_hi, rbuf[r, sl_hi] + hi))
                for sl, v in acc:
                    rbuf[r, sl] = v
            return c

        lax.fori_loop(0, CHUNK, add_row, 0)

    pecopy(0, 0).start()
    pecopy(1, 1).start()
    for t0 in range(LOOKAHEAD):
        gcopy(t0, t0 % NRING).start()

    def body(i, carry):
        for u in range(N_BODY):
            t = N_BODY * i + u
            nslot = (u + LOOKAHEAD) % NRING
            if u < NRING - LOOKAHEAD:
                @pl.when(i >= 1)
                def _ws():
                    scopy(t - (NRING - LOOKAHEAD), nslot).wait()
            else:
                scopy(t - (NRING - LOOKAHEAD), nslot).wait()
            if u >= N_BODY - LOOKAHEAD:
                @pl.when(i + 1 < N_ITERS)
                def _g():
                    gcopy(t + LOOKAHEAD, nslot).start()
            else:
                gcopy(t + LOOKAHEAD, nslot).start()

            if u == 0:
                @pl.when(i >= 1)
                def _p1():
                    pecopy(2 * i + 1, 1).start()
                pecopy(2 * i, 0).wait()
            elif u == 4:
                @pl.when(i + 1 < N_ITERS)
                def _p0():
                    pecopy(2 * i + 2, 0).start()
                pecopy(2 * i + 1, 1).wait()

            gcopy(t, u % NRING).wait()
            add_chunk(rows[u % NRING], pes[0] if u < 4 else pes[1])
            scopy(t, u % NRING).start()
        return carry

    lax.fori_loop(0, N_ITERS, body, 0)
    for tl in range(N_STEPS - (NRING - LOOKAHEAD), N_STEPS):
        scopy(tl, tl % NRING).wait()


def kernel(x, token_table):
    x2d = x.astype(jnp.int32)
    pe = jnp.asarray(_PE_PACKED)

    mesh = plsc.VectorSubcoreMesh(core_axis_name="c", subcore_axis_name="s")
    run = pl.kernel(
        _sc_body,
        out_type=jax.ShapeDtypeStruct((BATCH * SEQ_LEN, EMBED_DIM), jnp.float32),
        mesh=mesh,
        scratch_types=(
            [pltpu.VMEM((BATCH, POS_PER_WORKER), jnp.int32)]
            + [pltpu.VMEM((CHUNK, PE_WORDS), jnp.int32)] * 2
            + [pltpu.VMEM((CHUNK, EMBED_DIM), jnp.float32)] * NRING
            + [pltpu.SemaphoreType.DMA] * (2 * NRING + 2)
        ),
    )
    out = run(x2d, token_table, pe)
    return out.reshape(BATCH, SEQ_LEN, EMBED_DIM)
